# baseline (speedup 1.0000x reference)
import functools

import jax
import jax.numpy as jnp
from jax import lax
from jax.experimental import pallas as pl
from jax.experimental.pallas import tpu as pltpu
from jax.experimental.pallas import tpu_sc as plsc

N = 10000
E = 320000
N2 = 10240
CH = 128
NCHUNKS = E // CH
NWORKERS = 32
CHUNKS_BASE = NCHUNKS // NWORKERS
CHUNKS_EXTRA = NCHUNKS % NWORKERS
ROWS_PER_TILE = N2 // 16
CW = 16
F32 = jnp.float32


def _zero_buf(buf, nrow, ncol):
    def body(i, carry):
        for j in range(ncol // 16):
            buf[i, pl.ds(j * 16, 16)] = jnp.zeros((16,), F32)
        return carry
    lax.fori_loop(0, nrow, body, 0)


def _make_sc_agg(width, with_counts):
    mesh = plsc.VectorSubcoreMesh(core_axis_name="c", subcore_axis_name="s")
    nb = CHUNKS_BASE
    unroll = 4
    n_main = 72

    out_type = [jax.ShapeDtypeStruct((2, N2, width), F32)]
    scratch = [
        pltpu.VMEM_SHARED((N2, width), F32),
        pltpu.VMEM((CH, width), F32),
        pltpu.VMEM((CH, width), F32),
        pltpu.VMEM((2, CH), jnp.int32),
        pltpu.VMEM((2, CH), jnp.int32),
        pltpu.VMEM((2, CH), jnp.int32),
        pltpu.VMEM((2, CH), jnp.int32),
        pltpu.SemaphoreType.DMA,
        pltpu.SemaphoreType.DMA,
        pltpu.SemaphoreType.DMA,
        pltpu.SemaphoreType.DMA,
        pltpu.SemaphoreType.DMA,
        pltpu.SemaphoreType.DMA,
        pltpu.SemaphoreType.DMA,
    ]
    if with_counts:
        out_type.append(jax.ShapeDtypeStruct((2, N2, CW), F32))
        scratch.append(pltpu.VMEM_SHARED((N2, CW), F32))
        scratch.append(pltpu.VMEM((CH, CW), F32))

    @functools.partial(
        pl.kernel,
        out_type=tuple(out_type) if with_counts else out_type[0],
        mesh=mesh,
        scratch_types=scratch,
        compiler_params=pltpu.CompilerParams(use_tc_tiling_on_sc=False),
    )
    def sc_agg(table, ei2, *refs):
        if with_counts:
            (out, cout, acc, rows0, rows1, i0, i1, i2, i3,
             sg0, sg1, si0, si1, si2, si3, sc0, cacc, ones) = refs
        else:
            (out, acc, rows0, rows1, i0, i1, i2, i3,
             sg0, sg1, si0, si1, si2, si3, sc0) = refs
        c = lax.axis_index("c")
        s = lax.axis_index("s")
        wid = s * 2 + c
        r0 = s * ROWS_PER_TILE
        rows = (rows0, rows1)
        idx = (i0, i1, i2, i3)
        sg = (sg0, sg1)
        si = (si0, si1, si2, si3)
        start = wid * nb + jnp.minimum(wid, CHUNKS_EXTRA)

        def idx_start(k, slot):
            pltpu.async_copy(ei2.at[start + k], idx[slot], si[slot])

        def idx_wait(slot):
            pltpu.make_async_copy(ei2.at[0], idx[slot], si[slot]).wait()

        def gat_start(slot, buf):
            pltpu.async_copy(table.at[idx[slot].at[0]], rows[buf], sg[buf])

        def gat_wait(slot, buf):
            pltpu.make_async_copy(table.at[idx[slot].at[0]], rows[buf],
                                  sg[buf]).wait()

        def scat(slot, buf):
            pltpu.sync_copy(rows[buf], acc.at[idx[slot].at[1]], add=True)

        def cnt_start(slot):
            pltpu.async_copy(ones, cacc.at[idx[slot].at[1]], sc0, add=True)

        def cnt_wait(slot):
            pltpu.make_async_copy(ones, cacc.at[idx[slot].at[1]], sc0).wait()

        idx_start(0, 0)
        idx_start(1, 1)
        idx_start(2, 2)
        _zero_buf(rows0, CH, width)
        for b in range(ROWS_PER_TILE // CH):
            pltpu.sync_copy(rows0, acc.at[pl.ds(r0 + b * CH, CH)])
        if with_counts:
            _zero_buf(ones, CH, CW)
            for b in range(ROWS_PER_TILE // CH):
                pltpu.sync_copy(ones, cacc.at[pl.ds(r0 + b * CH, CH)])
            def fill_ones(i, carry):
                ones[i, pl.ds(0, 16)] = jnp.ones((16,), F32)
                return carry
            lax.fori_loop(0, CH, fill_ones, 0)
        plsc.subcore_barrier()

        idx_wait(0)
        gat_start(0, 0)
        idx_wait(1)
        gat_start(1, 1)

        def step(k, ks):
            gat_wait(ks % 4, ks % 2)
            scat(ks % 4, ks % 2)
            if with_counts:
                if ks >= 1:
                    cnt_wait((ks - 1) % 4)
                cnt_start(ks % 4)
            if ks + 3 < nb:
                idx_start(k + 3, (ks + 3) % 4)
            if ks + 2 < nb:
                idx_wait((ks + 2) % 4)
                gat_start((ks + 2) % 4, ks % 2)

        step(0, 0)

        def body(j, carry):
            k = 1 + j * unroll
            for u in range(unroll):
                step(k + u, 1 + u)
            return carry

        lax.fori_loop(0, n_main // unroll, body, 0)
        for u in range(nb - n_main - 1):
            step(n_main + 1 + u, n_main + 1 + u)
        if with_counts:
            cnt_wait((nb - 1) % 4)

        @pl.when(wid < CHUNKS_EXTRA)
        def _():
            pltpu.sync_copy(ei2.at[start + nb], i0)
            pltpu.async_copy(table.at[i0.at[0]], rows0, sg0).wait()
            pltpu.sync_copy(rows0, acc.at[i0.at[1]], add=True)
            if with_counts:
                pltpu.sync_copy(ones, cacc.at[i0.at[1]], add=True)

        plsc.subcore_barrier()
        pltpu.sync_copy(acc.at[pl.ds(r0, ROWS_PER_TILE)],
                        out.at[c, pl.ds(r0, ROWS_PER_TILE)])
        if with_counts:
            pltpu.sync_copy(cacc.at[pl.ds(r0, ROWS_PER_TILE)],
                            cout.at[c, pl.ds(r0, ROWS_PER_TILE)])

    return sc_agg


def _make_sc_agg_async(width):
    mesh = plsc.VectorSubcoreMesh(core_axis_name="c", subcore_axis_name="s")
    nb = CHUNKS_BASE
    rpt = N // 16

    @functools.partial(
        pl.kernel,
        out_type=jax.ShapeDtypeStruct((2, N, width), F32),
        mesh=mesh,
        scratch_types=[
            pltpu.VMEM_SHARED((N, width), F32),
            pltpu.VMEM((CH, width), F32),
            pltpu.VMEM((CH, width), F32),
            pltpu.VMEM((CH, width), F32),
            pltpu.VMEM((2, CH), jnp.int32),
            pltpu.VMEM((2, CH), jnp.int32),
            pltpu.VMEM((2, CH), jnp.int32),
            pltpu.VMEM((2, CH), jnp.int32),
            pltpu.SemaphoreType.DMA,
            pltpu.SemaphoreType.DMA,
            pltpu.SemaphoreType.DMA,
            pltpu.SemaphoreType.DMA,
            pltpu.SemaphoreType.DMA,
            pltpu.SemaphoreType.DMA,
            pltpu.SemaphoreType.DMA,
            pltpu.SemaphoreType.DMA,
            pltpu.SemaphoreType.DMA,
            pltpu.SemaphoreType.DMA,
        ],
        compiler_params=pltpu.CompilerParams(use_tc_tiling_on_sc=False),
    )
    def sc_agg(table, ei2, out, acc, r0b, r1b, r2b, i0, i1, i2, i3,
               sg0, sg1, sg2, ss0, ss1, ss2, si0, si1, si2, si3):
        c = lax.axis_index("c")
        s = lax.axis_index("s")
        wid = s * 2 + c
        r0 = s * rpt
        rows = (r0b, r1b, r2b)
        idx = (i0, i1, i2, i3)
        sg = (sg0, sg1, sg2)
        ss = (ss0, ss1, ss2)
        si = (si0, si1, si2, si3)
        start = wid * nb + jnp.minimum(wid, CHUNKS_EXTRA)

        def idx_start(k, slot):
            pltpu.async_copy(ei2.at[start + k], idx[slot], si[slot])

        def idx_wait(slot):
            pltpu.make_async_copy(ei2.at[0], idx[slot], si[slot]).wait()

        def gat_start(slot, buf):
            pltpu.async_copy(table.at[idx[slot].at[0]], rows[buf], sg[buf])

        def gat_wait(slot, buf):
            pltpu.make_async_copy(table.at[idx[slot].at[0]], rows[buf],
                                  sg[buf]).wait()

        def scat_start(slot, buf):
            pltpu.async_copy(rows[buf], acc.at[idx[slot].at[1]], ss[buf],
                             add=True)

        def scat_wait(slot, buf):
            pltpu.make_async_copy(rows[buf], acc.at[idx[slot].at[1]],
                                  ss[buf]).wait()

        idx_start(0, 0)
        idx_start(1, 1)
        idx_start(2, 2)
        idx_wait(0)
        gat_start(0, 0)
        idx_wait(1)
        gat_start(1, 1)
        _zero_buf(r2b, CH, width)
        for b in range(rpt // CH):
            pltpu.sync_copy(r2b, acc.at[pl.ds(r0 + b * CH, CH)])
        rem = rpt % CH
        pltpu.sync_copy(r2b.at[pl.ds(0, rem)],
                        acc.at[pl.ds(r0 + (rpt // CH) * CH, rem)])
        plsc.subcore_barrier()

        def step(k, ks):
            gat_wait(ks % 4, ks % 3)
            if ks >= 1:
                scat_wait((ks - 1) % 4, (ks - 1) % 3)
            if ks + 2 < nb:
                idx_wait((ks + 2) % 4)
                gat_start((ks + 2) % 4, (ks + 2) % 3)
            scat_start(ks % 4, ks % 3)
            if ks + 3 < nb:
                idx_start(k + 3, (ks + 3) % 4)

        step(0, 0)

        def body(j, carry):
            k = 1 + j * 12
            for u in range(12):
                step(k + u, 1 + u)
            return carry

        lax.fori_loop(0, 6, body, 0)
        for u in range(5):
            step(73 + u, 73 + u)
        scat_wait(77 % 4, 77 % 3)

        @pl.when(wid < CHUNKS_EXTRA)
        def _():
            pltpu.sync_copy(ei2.at[start + nb], i0)
            pltpu.async_copy(table.at[i0.at[0]], r0b, sg0).wait()
            pltpu.sync_copy(r0b, acc.at[i0.at[1]], add=True)

        plsc.subcore_barrier()
        pltpu.sync_copy(acc.at[pl.ds(r0, rpt)], out.at[c, pl.ds(r0, rpt)])

    return sc_agg


_sc_agg1 = _make_sc_agg(128, True)
_sc_agg128 = _make_sc_agg_async(128)
_sc_agg64 = _make_sc_agg_async(64)

_R = 1000
_GRID = N // _R


def _inv_cnt(cnt_ref):
    cnt = cnt_ref[0, :, 0:1] + cnt_ref[1, :, 0:1]
    return 1.0 / jnp.maximum(cnt, 1.0)


def _body1(agg_ref, cnt_ref, x_ref, wl_ref, wr_ref, b_ref, h_ref):
    mean = (agg_ref[0] + agg_ref[1]) * _inv_cnt(cnt_ref)
    h = (jnp.dot(mean, wl_ref[...], preferred_element_type=F32)
         + jnp.dot(x_ref[...], wr_ref[...], preferred_element_type=F32)
         + b_ref[...])
    h_ref[...] = jnp.maximum(h, 0.0)


def _dense1(aggc, cnt, x, Wl, Wr, b):
    return pl.pallas_call(
        _body1,
        grid=(_GRID,),
        in_specs=[
            pl.BlockSpec((2, _R, 128), lambda i: (0, i, 0)),
            pl.BlockSpec((2, _R, CW), lambda i: (0, i, 0)),
            pl.BlockSpec((_R, 128), lambda i: (i, 0)),
            pl.BlockSpec((128, 128), lambda i: (0, 0)),
            pl.BlockSpec((128, 128), lambda i: (0, 0)),
            pl.BlockSpec((1, 128), lambda i: (0, 0)),
        ],
        out_specs=pl.BlockSpec((_R, 128), lambda i: (i, 0)),
        out_shape=jax.ShapeDtypeStruct((N, 128), F32),
    )(aggc, cnt, x, Wl, Wr, b)


def _body2(agg_ref, cnt_ref, h1_ref, wl_ref, wr_ref, b_ref, wl3_ref,
           h2_ref, p3_ref):
    mean = (agg_ref[0] + agg_ref[1]) * _inv_cnt(cnt_ref)
    h2 = (jnp.dot(mean, wl_ref[...], preferred_element_type=F32)
          + jnp.dot(h1_ref[...], wr_ref[...], preferred_element_type=F32)
          + b_ref[...])
    h2 = jnp.maximum(h2, 0.0)
    h2_ref[...] = h2
    p3_ref[...] = jnp.dot(h2, wl3_ref[...], preferred_element_type=F32)


def _dense2(agg2, cnt, h1, Wl, Wr, b, Wl3):
    return pl.pallas_call(
        _body2,
        grid=(_GRID,),
        in_specs=[
            pl.BlockSpec((2, _R, 128), lambda i: (0, i, 0)),
            pl.BlockSpec((2, _R, CW), lambda i: (0, i, 0)),
            pl.BlockSpec((_R, 128), lambda i: (i, 0)),
            pl.BlockSpec((128, 128), lambda i: (0, 0)),
            pl.BlockSpec((128, 128), lambda i: (0, 0)),
            pl.BlockSpec((1, 128), lambda i: (0, 0)),
            pl.BlockSpec((128, 64), lambda i: (0, 0)),
        ],
        out_specs=[
            pl.BlockSpec((_R, 128), lambda i: (i, 0)),
            pl.BlockSpec((_R, 64), lambda i: (i, 0)),
        ],
        out_shape=[
            jax.ShapeDtypeStruct((N, 128), F32),
            jax.ShapeDtypeStruct((N, 64), F32),
        ],
    )(agg2, cnt, h1, Wl, Wr, b, Wl3)


def _body3(agg_ref, cnt_ref, h2_ref, wr_ref, b_ref, o_ref):
    mean = (agg_ref[0] + agg_ref[1]) * _inv_cnt(cnt_ref)
    o_ref[...] = (mean
                  + jnp.dot(h2_ref[...], wr_ref[...], preferred_element_type=F32)
                  + b_ref[...])


def _dense3(agg3, cnt, h2, Wr, b):
    return pl.pallas_call(
        _body3,
        grid=(_GRID,),
        in_specs=[
            pl.BlockSpec((2, _R, 64), lambda i: (0, i, 0)),
            pl.BlockSpec((2, _R, CW), lambda i: (0, i, 0)),
            pl.BlockSpec((_R, 128), lambda i: (i, 0)),
            pl.BlockSpec((128, 64), lambda i: (0, 0)),
            pl.BlockSpec((1, 64), lambda i: (0, 0)),
        ],
        out_specs=pl.BlockSpec((_R, 64), lambda i: (i, 0)),
        out_shape=jax.ShapeDtypeStruct((N, 64), F32),
    )(agg3, cnt, h2, Wr, b)


def kernel(x, edge_index, Wl1, Wr1, b1, Wl2, Wr2, b2, Wl3, Wr3, b3):
    ei2 = jnp.swapaxes(edge_index.astype(jnp.int32).reshape(2, NCHUNKS, CH),
                       0, 1)
    agg1, cnt = _sc_agg1(x, ei2)
    h1 = _dense1(agg1, cnt, x, Wl1, Wr1, b1.reshape(1, -1))
    agg2 = _sc_agg128(h1, ei2)
    h2, p3 = _dense2(agg2, cnt, h1, Wl2, Wr2, b2.reshape(1, -1), Wl3)
    agg3 = _sc_agg64(p3, ei2)
    return _dense3(agg3, cnt, h2, Wr3, b3.reshape(1, -1))


# speedup vs baseline: 1.0075x; 1.0075x over previous
import functools

import jax
import jax.numpy as jnp
from jax import lax
from jax.experimental import pallas as pl
from jax.experimental.pallas import tpu as pltpu
from jax.experimental.pallas import tpu_sc as plsc

N = 10000
E = 320000
N2 = 10240
CH = 128
NCHUNKS = E // CH
NWORKERS = 32
CHUNKS_BASE = NCHUNKS // NWORKERS
CHUNKS_EXTRA = NCHUNKS % NWORKERS
ROWS_PER_TILE = N2 // 16
CW = 16
F32 = jnp.float32


def _zero_buf(buf, nrow, ncol):
    def body(i, carry):
        for j in range(ncol // 16):
            buf[i, pl.ds(j * 16, 16)] = jnp.zeros((16,), F32)
        return carry
    lax.fori_loop(0, nrow, body, 0)


def _make_sc_agg(width, with_counts):
    mesh = plsc.VectorSubcoreMesh(core_axis_name="c", subcore_axis_name="s")
    nb = CHUNKS_BASE
    unroll = 4
    n_main = 72

    out_type = [jax.ShapeDtypeStruct((2, N2, width), F32)]
    scratch = [
        pltpu.VMEM_SHARED((N2, width), F32),
        pltpu.VMEM((CH, width), F32),
        pltpu.VMEM((CH, width), F32),
        pltpu.VMEM((2, CH), jnp.int32),
        pltpu.VMEM((2, CH), jnp.int32),
        pltpu.VMEM((2, CH), jnp.int32),
        pltpu.VMEM((2, CH), jnp.int32),
        pltpu.SemaphoreType.DMA,
        pltpu.SemaphoreType.DMA,
        pltpu.SemaphoreType.DMA,
        pltpu.SemaphoreType.DMA,
        pltpu.SemaphoreType.DMA,
        pltpu.SemaphoreType.DMA,
        pltpu.SemaphoreType.DMA,
    ]
    if with_counts:
        out_type.append(jax.ShapeDtypeStruct((2, N2, CW), F32))
        scratch.append(pltpu.VMEM_SHARED((N2, CW), F32))
        scratch.append(pltpu.VMEM((CH, CW), F32))

    @functools.partial(
        pl.kernel,
        out_type=tuple(out_type) if with_counts else out_type[0],
        mesh=mesh,
        scratch_types=scratch,
        compiler_params=pltpu.CompilerParams(use_tc_tiling_on_sc=False),
    )
    def sc_agg(table, ei2, *refs):
        if with_counts:
            (out, cout, acc, rows0, rows1, i0, i1, i2, i3,
             sg0, sg1, si0, si1, si2, si3, sc0, cacc, ones) = refs
        else:
            (out, acc, rows0, rows1, i0, i1, i2, i3,
             sg0, sg1, si0, si1, si2, si3, sc0) = refs
        c = lax.axis_index("c")
        s = lax.axis_index("s")
        wid = s * 2 + c
        r0 = s * ROWS_PER_TILE
        rows = (rows0, rows1)
        idx = (i0, i1, i2, i3)
        sg = (sg0, sg1)
        si = (si0, si1, si2, si3)
        start = wid * nb + jnp.minimum(wid, CHUNKS_EXTRA)

        def idx_start(k, slot):
            pltpu.async_copy(ei2.at[start + k], idx[slot], si[slot])

        def idx_wait(slot):
            pltpu.make_async_copy(ei2.at[0], idx[slot], si[slot]).wait()

        def gat_start(slot, buf):
            pltpu.async_copy(table.at[idx[slot].at[0]], rows[buf], sg[buf])

        def gat_wait(slot, buf):
            pltpu.make_async_copy(table.at[idx[slot].at[0]], rows[buf],
                                  sg[buf]).wait()

        def scat(slot, buf):
            pltpu.sync_copy(rows[buf], acc.at[idx[slot].at[1]], add=True)

        def cnt_start(slot):
            pltpu.async_copy(ones, cacc.at[idx[slot].at[1]], sc0, add=True)

        def cnt_wait(slot):
            pltpu.make_async_copy(ones, cacc.at[idx[slot].at[1]], sc0).wait()

        idx_start(0, 0)
        idx_start(1, 1)
        idx_start(2, 2)
        _zero_buf(rows0, CH, width)
        for b in range(ROWS_PER_TILE // CH):
            pltpu.sync_copy(rows0, acc.at[pl.ds(r0 + b * CH, CH)])
        if with_counts:
            _zero_buf(ones, CH, CW)
            for b in range(ROWS_PER_TILE // CH):
                pltpu.sync_copy(ones, cacc.at[pl.ds(r0 + b * CH, CH)])
            def fill_ones(i, carry):
                ones[i, pl.ds(0, 16)] = jnp.ones((16,), F32)
                return carry
            lax.fori_loop(0, CH, fill_ones, 0)
        plsc.subcore_barrier()

        idx_wait(0)
        gat_start(0, 0)
        idx_wait(1)
        gat_start(1, 1)

        def step(k, ks):
            gat_wait(ks % 4, ks % 2)
            scat(ks % 4, ks % 2)
            if with_counts:
                if ks >= 1:
                    cnt_wait((ks - 1) % 4)
                cnt_start(ks % 4)
            if ks + 3 < nb:
                idx_start(k + 3, (ks + 3) % 4)
            if ks + 2 < nb:
                idx_wait((ks + 2) % 4)
                gat_start((ks + 2) % 4, ks % 2)

        step(0, 0)

        def body(j, carry):
            k = 1 + j * unroll
            for u in range(unroll):
                step(k + u, 1 + u)
            return carry

        lax.fori_loop(0, n_main // unroll, body, 0)
        for u in range(nb - n_main - 1):
            step(n_main + 1 + u, n_main + 1 + u)
        if with_counts:
            cnt_wait((nb - 1) % 4)

        @pl.when(wid < CHUNKS_EXTRA)
        def _():
            pltpu.sync_copy(ei2.at[start + nb], i0)
            pltpu.async_copy(table.at[i0.at[0]], rows0, sg0).wait()
            pltpu.sync_copy(rows0, acc.at[i0.at[1]], add=True)
            if with_counts:
                pltpu.sync_copy(ones, cacc.at[i0.at[1]], add=True)

        plsc.subcore_barrier()
        pltpu.sync_copy(acc.at[pl.ds(r0, ROWS_PER_TILE)],
                        out.at[c, pl.ds(r0, ROWS_PER_TILE)])
        if with_counts:
            pltpu.sync_copy(cacc.at[pl.ds(r0, ROWS_PER_TILE)],
                            cout.at[c, pl.ds(r0, ROWS_PER_TILE)])

    return sc_agg


def _make_sc_agg_async(width):
    mesh = plsc.VectorSubcoreMesh(core_axis_name="c", subcore_axis_name="s")
    nb = CHUNKS_BASE
    rpt = N // 16

    @functools.partial(
        pl.kernel,
        out_type=jax.ShapeDtypeStruct((2, N, width), F32),
        mesh=mesh,
        scratch_types=[
            pltpu.VMEM_SHARED((N, width), F32),
            pltpu.VMEM((CH, width), F32),
            pltpu.VMEM((CH, width), F32),
            pltpu.VMEM((CH, width), F32),
            pltpu.VMEM((2, CH), jnp.int32),
            pltpu.VMEM((2, CH), jnp.int32),
            pltpu.VMEM((2, CH), jnp.int32),
            pltpu.VMEM((2, CH), jnp.int32),
            pltpu.SemaphoreType.DMA,
            pltpu.SemaphoreType.DMA,
            pltpu.SemaphoreType.DMA,
            pltpu.SemaphoreType.DMA,
            pltpu.SemaphoreType.DMA,
            pltpu.SemaphoreType.DMA,
            pltpu.SemaphoreType.DMA,
            pltpu.SemaphoreType.DMA,
            pltpu.SemaphoreType.DMA,
            pltpu.SemaphoreType.DMA,
        ],
        compiler_params=pltpu.CompilerParams(use_tc_tiling_on_sc=False),
    )
    def sc_agg(table, ei2, out, acc, r0b, r1b, r2b, i0, i1, i2, i3,
               sg0, sg1, sg2, ss0, ss1, ss2, si0, si1, si2, si3):
        c = lax.axis_index("c")
        s = lax.axis_index("s")
        wid = s * 2 + c
        r0 = s * rpt
        rows = (r0b, r1b, r2b)
        idx = (i0, i1, i2, i3)
        sg = (sg0, sg1, sg2)
        ss = (ss0, ss1, ss2)
        si = (si0, si1, si2, si3)
        start = wid * nb + jnp.minimum(wid, CHUNKS_EXTRA)

        def idx_start(k, slot):
            pltpu.async_copy(ei2.at[start + k], idx[slot], si[slot])

        def idx_wait(slot):
            pltpu.make_async_copy(ei2.at[0], idx[slot], si[slot]).wait()

        def gat_start(slot, buf):
            pltpu.async_copy(table.at[idx[slot].at[0]], rows[buf], sg[buf])

        def gat_wait(slot, buf):
            pltpu.make_async_copy(table.at[idx[slot].at[0]], rows[buf],
                                  sg[buf]).wait()

        def scat_start(slot, buf):
            pltpu.async_copy(rows[buf], acc.at[idx[slot].at[1]], ss[buf],
                             add=True)

        def scat_wait(slot, buf):
            pltpu.make_async_copy(rows[buf], acc.at[idx[slot].at[1]],
                                  ss[buf]).wait()

        idx_start(0, 0)
        idx_start(1, 1)
        idx_start(2, 2)
        idx_wait(0)
        gat_start(0, 0)
        idx_wait(1)
        gat_start(1, 1)
        _zero_buf(r2b, CH, width)
        for b in range(rpt // CH):
            pltpu.sync_copy(r2b, acc.at[pl.ds(r0 + b * CH, CH)])
        rem = rpt % CH
        pltpu.sync_copy(r2b.at[pl.ds(0, rem)],
                        acc.at[pl.ds(r0 + (rpt // CH) * CH, rem)])
        plsc.subcore_barrier()

        def step(k, ks):
            gat_wait(ks % 4, ks % 3)
            if ks >= 1:
                scat_wait((ks - 1) % 4, (ks - 1) % 3)
            scat_start(ks % 4, ks % 3)
            if ks + 3 < nb:
                idx_start(k + 3, (ks + 3) % 4)
            if ks + 2 < nb:
                idx_wait((ks + 2) % 4)
                gat_start((ks + 2) % 4, (ks + 2) % 3)

        step(0, 0)

        def body(j, carry):
            k = 1 + j * 12
            for u in range(12):
                step(k + u, 1 + u)
            return carry

        lax.fori_loop(0, 6, body, 0)
        for u in range(5):
            step(73 + u, 73 + u)
        scat_wait(77 % 4, 77 % 3)

        @pl.when(wid < CHUNKS_EXTRA)
        def _():
            pltpu.sync_copy(ei2.at[start + nb], i0)
            pltpu.async_copy(table.at[i0.at[0]], r0b, sg0).wait()
            pltpu.sync_copy(r0b, acc.at[i0.at[1]], add=True)

        plsc.subcore_barrier()
        pltpu.sync_copy(acc.at[pl.ds(r0, rpt)], out.at[c, pl.ds(r0, rpt)])

    return sc_agg


_sc_agg1 = _make_sc_agg(128, True)
_sc_agg128 = _make_sc_agg_async(128)
_sc_agg64 = _make_sc_agg_async(64)

_R = 1000
_GRID = N // _R


def _inv_cnt(cnt_ref):
    cnt = cnt_ref[0, :, 0:1] + cnt_ref[1, :, 0:1]
    return 1.0 / jnp.maximum(cnt, 1.0)


def _body1(agg_ref, cnt_ref, x_ref, wl_ref, wr_ref, b_ref, h_ref):
    mean = (agg_ref[0] + agg_ref[1]) * _inv_cnt(cnt_ref)
    h = (jnp.dot(mean, wl_ref[...], preferred_element_type=F32)
         + jnp.dot(x_ref[...], wr_ref[...], preferred_element_type=F32)
         + b_ref[...])
    h_ref[...] = jnp.maximum(h, 0.0)


def _dense1(aggc, cnt, x, Wl, Wr, b):
    return pl.pallas_call(
        _body1,
        grid=(_GRID,),
        in_specs=[
            pl.BlockSpec((2, _R, 128), lambda i: (0, i, 0)),
            pl.BlockSpec((2, _R, CW), lambda i: (0, i, 0)),
            pl.BlockSpec((_R, 128), lambda i: (i, 0)),
            pl.BlockSpec((128, 128), lambda i: (0, 0)),
            pl.BlockSpec((128, 128), lambda i: (0, 0)),
            pl.BlockSpec((1, 128), lambda i: (0, 0)),
        ],
        out_specs=pl.BlockSpec((_R, 128), lambda i: (i, 0)),
        out_shape=jax.ShapeDtypeStruct((N, 128), F32),
    )(aggc, cnt, x, Wl, Wr, b)


def _body2(agg_ref, cnt_ref, h1_ref, wl_ref, wr_ref, b_ref, wl3_ref,
           h2_ref, p3_ref):
    mean = (agg_ref[0] + agg_ref[1]) * _inv_cnt(cnt_ref)
    h2 = (jnp.dot(mean, wl_ref[...], preferred_element_type=F32)
          + jnp.dot(h1_ref[...], wr_ref[...], preferred_element_type=F32)
          + b_ref[...])
    h2 = jnp.maximum(h2, 0.0)
    h2_ref[...] = h2
    p3_ref[...] = jnp.dot(h2, wl3_ref[...], preferred_element_type=F32)


def _dense2(agg2, cnt, h1, Wl, Wr, b, Wl3):
    return pl.pallas_call(
        _body2,
        grid=(_GRID,),
        in_specs=[
            pl.BlockSpec((2, _R, 128), lambda i: (0, i, 0)),
            pl.BlockSpec((2, _R, CW), lambda i: (0, i, 0)),
            pl.BlockSpec((_R, 128), lambda i: (i, 0)),
            pl.BlockSpec((128, 128), lambda i: (0, 0)),
            pl.BlockSpec((128, 128), lambda i: (0, 0)),
            pl.BlockSpec((1, 128), lambda i: (0, 0)),
            pl.BlockSpec((128, 64), lambda i: (0, 0)),
        ],
        out_specs=[
            pl.BlockSpec((_R, 128), lambda i: (i, 0)),
            pl.BlockSpec((_R, 64), lambda i: (i, 0)),
        ],
        out_shape=[
            jax.ShapeDtypeStruct((N, 128), F32),
            jax.ShapeDtypeStruct((N, 64), F32),
        ],
    )(agg2, cnt, h1, Wl, Wr, b, Wl3)


def _body3(agg_ref, cnt_ref, h2_ref, wr_ref, b_ref, o_ref):
    mean = (agg_ref[0] + agg_ref[1]) * _inv_cnt(cnt_ref)
    o_ref[...] = (mean
                  + jnp.dot(h2_ref[...], wr_ref[...], preferred_element_type=F32)
                  + b_ref[...])


def _dense3(agg3, cnt, h2, Wr, b):
    return pl.pallas_call(
        _body3,
        grid=(_GRID,),
        in_specs=[
            pl.BlockSpec((2, _R, 64), lambda i: (0, i, 0)),
            pl.BlockSpec((2, _R, CW), lambda i: (0, i, 0)),
            pl.BlockSpec((_R, 128), lambda i: (i, 0)),
            pl.BlockSpec((128, 64), lambda i: (0, 0)),
            pl.BlockSpec((1, 64), lambda i: (0, 0)),
        ],
        out_specs=pl.BlockSpec((_R, 64), lambda i: (i, 0)),
        out_shape=jax.ShapeDtypeStruct((N, 64), F32),
    )(agg3, cnt, h2, Wr, b)


def kernel(x, edge_index, Wl1, Wr1, b1, Wl2, Wr2, b2, Wl3, Wr3, b3):
    ei2 = jnp.swapaxes(edge_index.astype(jnp.int32).reshape(2, NCHUNKS, CH),
                       0, 1)
    agg1, cnt = _sc_agg1(x, ei2)
    h1 = _dense1(agg1, cnt, x, Wl1, Wr1, b1.reshape(1, -1))
    agg2 = _sc_agg128(h1, ei2)
    h2, p3 = _dense2(agg2, cnt, h1, Wl2, Wr2, b2.reshape(1, -1), Wl3)
    agg3 = _sc_agg64(p3, ei2)
    return _dense3(agg3, cnt, h2, Wr3, b3.reshape(1, -1))
